# baseline (speedup 1.0000x reference)
import functools

import jax
import jax.numpy as jnp
from jax import lax
from jax.experimental import pallas as pl
from jax.experimental.pallas import tpu as pltpu
from jax.experimental.pallas import tpu_sc as plsc

_LANES = 16


def _make_sc_kernel(B, V, NC, NS):
    NW = NC * NS
    bpw = B // NW

    mesh = plsc.VectorSubcoreMesh(core_axis_name="c", subcore_axis_name="s")

    @functools.partial(
        pl.kernel,
        mesh=mesh,
        out_type=jax.ShapeDtypeStruct((B,), jnp.float32),
        compiler_params=pltpu.CompilerParams(needs_layout_passes=False),
        scratch_types=[
            pltpu.VMEM((V,), jnp.float32),
            pltpu.VMEM((2 * bpw,), jnp.int32),
            pltpu.VMEM((bpw,), jnp.float32),
            pltpu.VMEM((bpw,), jnp.float32),
        ],
    )
    def run(x_hbm, ids_hbm, bias_hbm, out_hbm, table_v, ids_v, x_v, out_v):
        wid = lax.axis_index("s") * NC + lax.axis_index("c")
        base = wid * bpw
        pltpu.sync_copy(bias_hbm, table_v)
        pltpu.sync_copy(ids_hbm.at[pl.ds(base * 2, 2 * bpw)], ids_v)
        pltpu.sync_copy(x_hbm.at[pl.ds(base, bpw)], x_v)

        lane = lax.iota(jnp.int32, _LANES)

        def body(j, carry):
            ridx = j * _LANES + lane
            uid = plsc.load_gather(ids_v, [ridx * 2])
            bv = plsc.load_gather(table_v, [uid])
            xv = plsc.load_gather(x_v, [ridx])
            plsc.store_scatter(out_v, [ridx], xv + bv)
            return carry

        lax.fori_loop(0, bpw // _LANES, body, 0)
        pltpu.sync_copy(out_v, out_hbm.at[pl.ds(base, bpw)])

    return run


def kernel(x, user_ids, bias):
    B = x.shape[0]
    V = bias.shape[0]
    info = plsc.get_sparse_core_info()
    NC, NS = info.num_cores, info.num_subcores

    run = _make_sc_kernel(B, V, NC, NS)
    out = run(
        x.reshape(B).astype(jnp.float32),
        user_ids.reshape(2 * B).astype(jnp.int32),
        bias.astype(jnp.float32),
    )
    return out.reshape(B, 1)


# speedup vs baseline: 1.4283x; 1.4283x over previous
import functools

import jax
import jax.numpy as jnp
from jax import lax
from jax.experimental import pallas as pl
from jax.experimental.pallas import tpu as pltpu
from jax.experimental.pallas import tpu_sc as plsc

_LANES = 16


def _make_sc_gather(B, V, NC, NS):
    NW = NC * NS
    bpw = B // NW

    mesh = plsc.VectorSubcoreMesh(core_axis_name="c", subcore_axis_name="s")

    @functools.partial(
        pl.kernel,
        mesh=mesh,
        out_type=jax.ShapeDtypeStruct((B,), jnp.float32),
        compiler_params=pltpu.CompilerParams(needs_layout_passes=False),
        scratch_types=[
            pltpu.VMEM((V,), jnp.float32),
            pltpu.VMEM((bpw,), jnp.int32),
            pltpu.VMEM((bpw,), jnp.float32),
        ],
    )
    def run(uid_hbm, bias_hbm, out_hbm, table_v, uid_v, out_v):
        wid = lax.axis_index("s") * NC + lax.axis_index("c")
        base = wid * bpw
        pltpu.sync_copy(bias_hbm, table_v)
        pltpu.sync_copy(uid_hbm.at[pl.ds(base, bpw)], uid_v)

        lane = lax.iota(jnp.int32, _LANES)

        def body(j, carry):
            ridx = j * _LANES + lane
            uid = plsc.load_gather(uid_v, [ridx])
            bv = plsc.load_gather(table_v, [uid])
            plsc.store_scatter(out_v, [ridx], bv)
            return carry

        lax.fori_loop(0, bpw // _LANES, body, 0)
        pltpu.sync_copy(out_v, out_hbm.at[pl.ds(base, bpw)])

    return run


def kernel(x, user_ids, bias):
    B = x.shape[0]
    V = bias.shape[0]
    info = plsc.get_sparse_core_info()
    NC, NS = info.num_cores, info.num_subcores

    run = _make_sc_gather(B, V, NC, NS)
    gathered = run(user_ids[:, 0], bias)
    return x + gathered[:, None]


# speedup vs baseline: 1.4980x; 1.0488x over previous
import functools

import jax
import jax.numpy as jnp
from jax import lax
from jax.experimental import pallas as pl
from jax.experimental.pallas import tpu as pltpu
from jax.experimental.pallas import tpu_sc as plsc

_LANES = 16


def _make_sc_gather(B, V, NC, NS):
    NW = NC * NS
    bpw = B // NW

    mesh = plsc.VectorSubcoreMesh(core_axis_name="c", subcore_axis_name="s")

    @functools.partial(
        pl.kernel,
        mesh=mesh,
        out_type=jax.ShapeDtypeStruct((B,), jnp.float32),
        compiler_params=pltpu.CompilerParams(needs_layout_passes=False),
        scratch_types=[
            pltpu.VMEM((bpw,), jnp.int32),
            pltpu.VMEM((bpw,), jnp.float32),
            pltpu.SemaphoreType.DMA,
        ],
    )
    def run(uid_hbm, bias_hbm, out_hbm, uid_v, out_v, sem):
        wid = lax.axis_index("s") * NC + lax.axis_index("c")
        base = wid * bpw
        pltpu.sync_copy(uid_hbm.at[pl.ds(base, bpw)], uid_v)
        pltpu.async_copy(bias_hbm.at[uid_v], out_v, sem).wait()
        pltpu.sync_copy(out_v, out_hbm.at[pl.ds(base, bpw)])

    return run


def kernel(x, user_ids, bias):
    B = x.shape[0]
    V = bias.shape[0]
    info = plsc.get_sparse_core_info()
    NC, NS = info.num_cores, info.num_subcores

    run = _make_sc_gather(B, V, NC, NS)
    gathered = run(user_ids[:, 0], bias)
    return x + gathered[:, None]
